# baseline (speedup 1.0000x reference)
import functools

import jax
import jax.numpy as jnp
from jax import lax
from jax.experimental import pallas as pl
from jax.experimental.pallas import tpu as pltpu
from jax.experimental.pallas import tpu_sc as plsc

_NUM_FIELDS = 26
_VOCAB = 100000
_EMB_DIM = 32
_LANES = 16

_info = plsc.get_sparse_core_info()
_NC, _NS = _info.num_cores, _info.num_subcores
_NW = _NC * _NS


def _make_sc_gather(total_rows: int, chunk: int):
  rows_per_w = total_rows // _NW
  n_chunks = rows_per_w // chunk
  assert rows_per_w % chunk == 0 and chunk % _NUM_FIELDS == 0

  mesh = plsc.VectorSubcoreMesh(core_axis_name="c", subcore_axis_name="s")

  @functools.partial(
      pl.kernel,
      mesh=mesh,
      out_type=jax.ShapeDtypeStruct((total_rows, _EMB_DIM), jnp.float32),
      compiler_params=pltpu.CompilerParams(use_tc_tiling_on_sc=False),
      scratch_types=[
          pltpu.VMEM((chunk,), jnp.int32),
          pltpu.VMEM((chunk,), jnp.int32),
          pltpu.VMEM((chunk,), jnp.int32),
          pltpu.VMEM((chunk, _EMB_DIM), jnp.float32),
          pltpu.SemaphoreType.DMA,
      ],
  )
  def gather_kernel(x_hbm, off_hbm, tab_hbm, out_hbm, off_v, xv, idxg, rows, sem):
    wid = lax.axis_index("s") * _NC + lax.axis_index("c")
    base = wid * rows_per_w

    pltpu.sync_copy(off_hbm, off_v)

    @pl.loop(0, n_chunks)
    def _chunk(c):
      row0 = base + c * chunk
      pltpu.sync_copy(x_hbm.at[pl.ds(row0, chunk)], xv)

      @pl.loop(0, chunk // _LANES)
      def _add(j):
        s = pl.ds(j * _LANES, _LANES)
        idxg[s] = xv[s] + off_v[s]

      pltpu.async_copy(tab_hbm.at[idxg], rows, sem).wait()
      pltpu.sync_copy(rows, out_hbm.at[pl.ds(row0, chunk)])

  return gather_kernel


def kernel(x, tables):
  batch = x.shape[0]
  total_rows = batch * _NUM_FIELDS
  x_flat = x.reshape(total_rows).astype(jnp.int32)
  tab_flat = tables.reshape(_NUM_FIELDS * _VOCAB, _EMB_DIM)
  chunk = 1664
  off = jnp.tile(
      jnp.arange(_NUM_FIELDS, dtype=jnp.int32) * _VOCAB, chunk // _NUM_FIELDS
  )
  out = _make_sc_gather(total_rows, chunk)(x_flat, off, tab_flat)
  return out.reshape(batch, _NUM_FIELDS * _EMB_DIM)


# speedup vs baseline: 1.0044x; 1.0044x over previous
import functools

import jax
import jax.numpy as jnp
from jax import lax
from jax.experimental import pallas as pl
from jax.experimental.pallas import tpu as pltpu
from jax.experimental.pallas import tpu_sc as plsc

_NUM_FIELDS = 26
_VOCAB = 100000
_EMB_DIM = 32
_LANES = 16

_info = plsc.get_sparse_core_info()
_NC, _NS = _info.num_cores, _info.num_subcores
_NW = _NC * _NS


def _make_sc_gather(total_rows: int, chunk: int):
  rows_per_w = total_rows // _NW
  n_chunks = rows_per_w // chunk
  assert rows_per_w % chunk == 0 and chunk % _NUM_FIELDS == 0

  mesh = plsc.VectorSubcoreMesh(core_axis_name="c", subcore_axis_name="s")

  @functools.partial(
      pl.kernel,
      mesh=mesh,
      out_type=jax.ShapeDtypeStruct((total_rows, _EMB_DIM), jnp.float32),
      compiler_params=pltpu.CompilerParams(use_tc_tiling_on_sc=False),
      scratch_types=[
          pltpu.VMEM((chunk,), jnp.int32),
          pltpu.VMEM((chunk,), jnp.int32),
          pltpu.VMEM((chunk,), jnp.int32),
          pltpu.VMEM((chunk,), jnp.int32),
          pltpu.VMEM((chunk, _EMB_DIM), jnp.float32),
          pltpu.VMEM((chunk, _EMB_DIM), jnp.float32),
          pltpu.SemaphoreType.DMA,
          pltpu.SemaphoreType.DMA,
          pltpu.SemaphoreType.DMA,
          pltpu.SemaphoreType.DMA,
      ],
  )
  def gather_kernel(x_hbm, off_hbm, tab_hbm, out_hbm,
                    off_v, xv, idxg0, idxg1, rows0, rows1, sg0, sg1, sw0, sw1):
    wid = lax.axis_index("s") * _NC + lax.axis_index("c")
    base = wid * rows_per_w

    pltpu.sync_copy(off_hbm, off_v)

    idxg = (idxg0, idxg1)
    rows = (rows0, rows1)
    sg = (sg0, sg1)
    sw = (sw0, sw1)

    def start_gather(c):
      b = c % 2
      pltpu.sync_copy(x_hbm.at[pl.ds(base + c * chunk, chunk)], xv)

      @pl.loop(0, chunk // _LANES)
      def _add(j):
        s = pl.ds(j * _LANES, _LANES)
        idxg[b][s] = xv[s] + off_v[s]

      return pltpu.async_copy(tab_hbm.at[idxg[b]], rows[b], sg[b])

    gather_d = [None, None]
    write_d = [None, None]
    gather_d[0] = start_gather(0)
    for c in range(n_chunks):
      b, nb = c % 2, (c + 1) % 2
      if c + 1 < n_chunks:
        if write_d[nb] is not None:
          write_d[nb].wait()
        gather_d[nb] = start_gather(c + 1)
      gather_d[b].wait()
      write_d[b] = pltpu.async_copy(
          rows[b], out_hbm.at[pl.ds(base + c * chunk, chunk)], sw[b])
    for d in write_d:
      if d is not None:
        d.wait()

  return gather_kernel


def kernel(x, tables):
  batch = x.shape[0]
  total_rows = batch * _NUM_FIELDS
  x_flat = x.reshape(total_rows).astype(jnp.int32)
  tab_flat = tables.reshape(_NUM_FIELDS * _VOCAB, _EMB_DIM)
  chunk = 1664
  off = jnp.tile(
      jnp.arange(_NUM_FIELDS, dtype=jnp.int32) * _VOCAB, chunk // _NUM_FIELDS
  )
  out = _make_sc_gather(total_rows, chunk)(x_flat, off, tab_flat)
  return out.reshape(batch, _NUM_FIELDS * _EMB_DIM)


# speedup vs baseline: 3.4436x; 3.4287x over previous
import functools

import jax
import jax.numpy as jnp
from jax import lax
from jax.experimental import pallas as pl
from jax.experimental.pallas import tpu as pltpu
from jax.experimental.pallas import tpu_sc as plsc

_NUM_FIELDS = 26
_VOCAB = 100000
_EMB_DIM = 32
_LANES = 16
_BCHUNK = 8192

_info = plsc.get_sparse_core_info()
_NC, _NS = _info.num_cores, _info.num_subcores
_NW = _NC * _NS


def _make_sc_gather(batch: int):
  n_bchunks = batch // _BCHUNK
  assert batch % _BCHUNK == 0 and _NW == _EMB_DIM

  mesh = plsc.VectorSubcoreMesh(core_axis_name="c", subcore_axis_name="s")

  @functools.partial(
      pl.kernel,
      mesh=mesh,
      out_type=jax.ShapeDtypeStruct((_NUM_FIELDS * _EMB_DIM, batch), jnp.float32),
      compiler_params=pltpu.CompilerParams(
          use_tc_tiling_on_sc=True, needs_layout_passes=False),
      scratch_types=[
          pltpu.VMEM((_VOCAB,), jnp.float32),
          pltpu.VMEM((_BCHUNK,), jnp.int32),
          pltpu.VMEM((_BCHUNK,), jnp.float32),
      ],
  )
  def gather_kernel(xt_hbm, tt_hbm, out_hbm, row_v, idx_v, out_v):
    w = lax.axis_index("s") * _NC + lax.axis_index("c")

    for f in range(_NUM_FIELDS):
      pltpu.sync_copy(tt_hbm.at[f, w], row_v)
      for bc in range(n_bchunks):
        bsl = pl.ds(bc * _BCHUNK, _BCHUNK)
        pltpu.sync_copy(xt_hbm.at[f, bsl], idx_v)

        @pl.loop(0, _BCHUNK // _LANES)
        def _g(j):
          s = pl.ds(j * _LANES, _LANES)
          out_v[s] = plsc.load_gather(row_v, [idx_v[s]])

        pltpu.sync_copy(out_v, out_hbm.at[f * _EMB_DIM + w, bsl])

  return gather_kernel


def kernel(x, tables):
  batch = x.shape[0]
  x_t = x.T.astype(jnp.int32)
  tab_t = tables.transpose(0, 2, 1)
  out_t = _make_sc_gather(batch)(x_t, tab_t)
  return out_t.T
